# baseline (speedup 1.0000x reference)
import jax
import jax.numpy as jnp
import numpy as np
from jax.experimental import pallas as pl
from jax.experimental.pallas import tpu as pltpu

N = 2
C_IN = 256
C_MID = 256
H = 50
W = 84
STRIDE = 16
SCALES = (32.0, 64.0, 128.0, 256.0, 512.0)
RATIOS = (0.5, 1.0, 2.0)
A = len(SCALES) * len(RATIOS)
IMG_W = 1344
IMG_H = 800
PRE_NMS = 6000
POST_NMS = 1000
NMS_THRESH = 0.7
BBOX_XFORM_CLIP = float(np.log(1000.0 / 16.0))

NUM = A * H * W
LANES = 128
ROWS = 496
PADN = ROWS * LANES
OUT_ROWS = 1024

_NEG = -1e10


def _anchor_planes():
    scales = jnp.asarray(SCALES, jnp.float32)
    ratios = jnp.asarray(RATIOS, jnp.float32)
    h_ratios = jnp.sqrt(ratios)
    w_ratios = 1.0 / h_ratios
    ws = (w_ratios[:, None] * scales[None, :]).reshape(-1)
    hs = (h_ratios[:, None] * scales[None, :]).reshape(-1)
    base = jnp.stack([-ws, -hs, ws, hs], axis=1) / 2.0
    sx = jnp.arange(W, dtype=jnp.float32) * STRIDE
    sy = jnp.arange(H, dtype=jnp.float32) * STRIDE
    yy, xx = jnp.meshgrid(sy, sx, indexing="ij")
    shifts = jnp.stack([xx.reshape(-1), yy.reshape(-1), xx.reshape(-1), yy.reshape(-1)], axis=1)
    anchors = (shifts[:, None, :] + base[None, :, :]).reshape(-1, 4)
    wa = anchors[:, 2] - anchors[:, 0]
    ha = anchors[:, 3] - anchors[:, 1]
    cxa = anchors[:, 0] + 0.5 * wa
    cya = anchors[:, 1] + 0.5 * ha
    out = []
    for v in (wa, ha, cxa, cya):
        out.append(jnp.pad(v, (0, PADN - NUM)).reshape(ROWS, LANES))
    return out


def _nms_kernel(keys_ref, dx_ref, dy_ref, dw_ref, dh_ref,
                wa_ref, ha_ref, cxa_ref, cya_ref,
                out_ref,
                sw_ref, x1_ref, y1_ref, x2_ref, y2_ref, a2_ref):
    idx = jax.lax.broadcasted_iota(jnp.int32, (ROWS, LANES), 0) * LANES + \
        jax.lax.broadcasted_iota(jnp.int32, (ROWS, LANES), 1)

    for a in range(N):
        lg = keys_ref[a, :, :]

        u = jax.lax.bitcast_convert_type(lg, jnp.int32)
        key = jnp.where(u >= 0, u, jnp.int32(-2147483648) - u)

        def tbody(_, lohi, key=key):
            lo, hi = lohi
            mid = (lo >> 1) + (hi >> 1) + (lo & hi & 1)
            cnt = jnp.sum((key >= mid).astype(jnp.int32))
            big = cnt >= PRE_NMS
            return (jnp.where(big, mid, lo), jnp.where(big, hi, mid))

        lo, _ = jax.lax.fori_loop(
            0, 32, tbody, (jnp.int32(-2147483648), jnp.int32(2147483647)))
        tie = key == lo
        need = PRE_NMS - jnp.sum((key > lo).astype(jnp.int32))

        def ibody(_, lohi, tie=tie, need=need):
            ilo, ihi = lohi
            mid = (ilo + ihi) // 2
            cnt = jnp.sum((tie & (idx < mid)).astype(jnp.int32))
            small = cnt < need
            return (jnp.where(small, mid, ilo), jnp.where(small, ihi, mid))

        _, istar = jax.lax.fori_loop(0, 17, ibody, (jnp.int32(0), jnp.int32(PADN)))
        sel = (key > lo) | (tie & (idx < istar))

        wa = wa_ref[...]
        ha = ha_ref[...]
        dw = jnp.minimum(dw_ref[a, :, :], BBOX_XFORM_CLIP)
        dh = jnp.minimum(dh_ref[a, :, :], BBOX_XFORM_CLIP)
        pcx = dx_ref[a, :, :] * wa + cxa_ref[...]
        pcy = dy_ref[a, :, :] * ha + cya_ref[...]
        pw = jnp.exp(dw) * wa
        ph = jnp.exp(dh) * ha
        x1 = jnp.clip(pcx - 0.5 * pw, 0.0, float(IMG_W))
        y1 = jnp.clip(pcy - 0.5 * ph, 0.0, float(IMG_H))
        x2 = jnp.clip(pcx + 0.5 * pw, 0.0, float(IMG_W))
        y2 = jnp.clip(pcy + 0.5 * ph, 0.0, float(IMG_H))
        keep = ((x2 - x1) >= 0.0) & ((y2 - y1) >= 0.0)

        sw_ref[a, :, :] = jnp.where(sel & keep, lg, _NEG)
        x1_ref[a, :, :] = x1
        y1_ref[a, :, :] = y1
        x2_ref[a, :, :] = x2
        y2_ref[a, :, :] = y2
        a2_ref[a, :, :] = (x2 - x1) * (y2 - y1)

    lane = jax.lax.broadcasted_iota(jnp.int32, (1, LANES), 1)

    def body(i, carry):
        for a in range(N):
            sw = sw_ref[a, :, :]
            m = jnp.max(sw)
            valid = m > -1e9
            istar = jnp.min(jnp.where(sw == m, idx, jnp.int32(PADN)))
            oh = idx == istar
            ohf = oh.astype(jnp.float32)
            x1 = x1_ref[a, :, :]
            y1 = y1_ref[a, :, :]
            x2 = x2_ref[a, :, :]
            y2 = y2_ref[a, :, :]
            bx1 = jnp.sum(x1 * ohf)
            by1 = jnp.sum(y1 * ohf)
            bx2 = jnp.sum(x2 * ohf)
            by2 = jnp.sum(y2 * ohf)
            ba = (bx2 - bx1) * (by2 - by1)
            iw = jnp.maximum(jnp.minimum(bx2, x2) - jnp.maximum(bx1, x1), 0.0)
            ih = jnp.maximum(jnp.minimum(by2, y2) - jnp.maximum(by1, y1), 0.0)
            inter = iw * ih
            iou = inter / (ba + a2_ref[a, :, :] - inter + 1e-9)
            sup = (iou > NMS_THRESH) | oh
            sw_ref[a, :, :] = jnp.where(sup & valid, _NEG, sw)
            vf = jnp.where(valid, 1.0, 0.0)
            row = jnp.where(lane == 0, jnp.where(valid, bx1, 0.0),
                  jnp.where(lane == 1, jnp.where(valid, by1, 0.0),
                  jnp.where(lane == 2, jnp.where(valid, bx2, 0.0),
                  jnp.where(lane == 3, jnp.where(valid, by2, 0.0),
                  jnp.where(lane == 4, vf, 0.0)))))
            out_ref[a, pl.ds(i, 1), :] = row
        return carry

    jax.lax.fori_loop(0, POST_NMS, body, 0)


def _run_nms(logits_flat, breg_flat):
    padk = jnp.pad(logits_flat, ((0, 0), (0, PADN - NUM)),
                   constant_values=_NEG).reshape(N, ROWS, LANES)
    regs = []
    for j in range(4):
        regs.append(jnp.pad(breg_flat[:, :, j], ((0, 0), (0, PADN - NUM))
                            ).reshape(N, ROWS, LANES))
    wa, ha, cxa, cya = _anchor_planes()

    img_spec = pl.BlockSpec((N, ROWS, LANES), lambda: (0, 0, 0))
    cst_spec = pl.BlockSpec((ROWS, LANES), lambda: (0, 0))
    out = pl.pallas_call(
        _nms_kernel,
        grid=(),
        in_specs=[img_spec] * 5 + [cst_spec] * 4,
        out_specs=pl.BlockSpec((N, OUT_ROWS, LANES), lambda: (0, 0, 0)),
        out_shape=jax.ShapeDtypeStruct((N, OUT_ROWS, LANES), jnp.float32),
        scratch_shapes=[pltpu.VMEM((N, ROWS, LANES), jnp.float32)] * 6,
    )(padk, *regs, wa, ha, cxa, cya)

    boxes = out[:, :POST_NMS, :4]
    valid = out[:, :POST_NMS, 4] > 0.5
    return boxes, valid


COLS = 4224
CHUNK = 1408
K_IM2COL = 9 * C_IN


def _trunk_kernel(x_ref, w_ref, b_ref, wh_ref, bh_ref, out_ref):
    x = x_ref[0, :, :]
    t = jnp.dot(x[:, :C_IN], w_ref[:C_IN, :], preferred_element_type=jnp.float32)
    for k in range(1, 9):
        t = t + jnp.dot(x[:, k * C_IN:(k + 1) * C_IN], w_ref[k * C_IN:(k + 1) * C_IN, :],
                        preferred_element_type=jnp.float32)
    t = jax.nn.relu(t + b_ref[...])
    o = jnp.dot(t, wh_ref[...], preferred_element_type=jnp.float32)
    out_ref[0, :, :] = o + bh_ref[...]


def _trunk(features, conv_w, conv_b, cls_w, cls_b, bbox_w, bbox_b):
    fp = jnp.pad(features, ((0, 0), (0, 0), (1, 1), (1, 1)))
    taps = [fp[:, :, kh:kh + H, kw:kw + W] for kh in range(3) for kw in range(3)]
    x = jnp.stack(taps, axis=1)
    x = x.transpose(0, 3, 4, 1, 2).reshape(N, H * W, K_IM2COL)
    x = jnp.pad(x, ((0, 0), (0, COLS - H * W), (0, 0)))
    w = conv_w.transpose(2, 3, 1, 0).reshape(K_IM2COL, C_MID)
    wh = jnp.zeros((C_MID, 256), jnp.float32)
    wh = wh.at[:, :A].set(cls_w[:, :, 0, 0].T)
    wh = wh.at[:, 128:128 + 4 * A].set(bbox_w[:, :, 0, 0].T)
    bh = jnp.zeros((256,), jnp.float32)
    bh = bh.at[:A].set(cls_b)
    bh = bh.at[128:128 + 4 * A].set(bbox_b)
    b2 = conv_b.reshape(1, C_MID)
    bh2 = bh.reshape(1, 256)

    out = pl.pallas_call(
        _trunk_kernel,
        grid=(N, COLS // CHUNK),
        in_specs=[
            pl.BlockSpec((1, CHUNK, K_IM2COL), lambda i, j: (i, j, 0)),
            pl.BlockSpec((K_IM2COL, C_MID), lambda i, j: (0, 0)),
            pl.BlockSpec((1, C_MID), lambda i, j: (0, 0)),
            pl.BlockSpec((C_MID, 256), lambda i, j: (0, 0)),
            pl.BlockSpec((1, 256), lambda i, j: (0, 0)),
        ],
        out_specs=pl.BlockSpec((1, CHUNK, 256), lambda i, j: (i, j, 0)),
        out_shape=jax.ShapeDtypeStruct((N, COLS, 256), jnp.float32),
        compiler_params=pltpu.CompilerParams(
            dimension_semantics=("parallel", "arbitrary")),
    )(x, w, b2, wh, bh2)
    return out


def _conv2d(x, w, b):
    y = jax.lax.conv_general_dilated(
        x, w, (1, 1), "SAME", dimension_numbers=("NCHW", "OIHW", "NCHW"))
    return y + b[None, :, None, None]


def kernel(images, features, img_metas, conv_w, conv_b, cls_w, cls_b, bbox_w, bbox_b):
    trunk = _trunk(features, conv_w, conv_b, cls_w, cls_b, bbox_w, bbox_b)
    logits = trunk[:, :H * W, :A].transpose(0, 2, 1).reshape(N, A, H, W)

    logits_flat = trunk[:, :H * W, :A].reshape(N, NUM)
    breg_flat = trunk[:, :H * W, 128:128 + 4 * A].reshape(N, NUM, 4)

    scores_flat = jax.nn.sigmoid(logits_flat)
    boxes, valid = _run_nms(scores_flat, breg_flat)
    return boxes, valid, logits


# speedup vs baseline: 1.2689x; 1.2689x over previous
import jax
import jax.numpy as jnp
import numpy as np
from jax.experimental import pallas as pl
from jax.experimental.pallas import tpu as pltpu

N = 2
C_IN = 256
C_MID = 256
H = 50
W = 84
STRIDE = 16
SCALES = (32.0, 64.0, 128.0, 256.0, 512.0)
RATIOS = (0.5, 1.0, 2.0)
A = len(SCALES) * len(RATIOS)
IMG_W = 1344
IMG_H = 800
PRE_NMS = 6000
POST_NMS = 1000
NMS_THRESH = 0.7
BBOX_XFORM_CLIP = float(np.log(1000.0 / 16.0))

NUM = A * H * W
LANES = 128
ROWS = 496
PADN = ROWS * LANES
OUT_ROWS = 1024

_NEG = -1e10


def _anchor_planes():
    scales = jnp.asarray(SCALES, jnp.float32)
    ratios = jnp.asarray(RATIOS, jnp.float32)
    h_ratios = jnp.sqrt(ratios)
    w_ratios = 1.0 / h_ratios
    ws = (w_ratios[:, None] * scales[None, :]).reshape(-1)
    hs = (h_ratios[:, None] * scales[None, :]).reshape(-1)
    base = jnp.stack([-ws, -hs, ws, hs], axis=1) / 2.0
    sx = jnp.arange(W, dtype=jnp.float32) * STRIDE
    sy = jnp.arange(H, dtype=jnp.float32) * STRIDE
    yy, xx = jnp.meshgrid(sy, sx, indexing="ij")
    shifts = jnp.stack([xx.reshape(-1), yy.reshape(-1), xx.reshape(-1), yy.reshape(-1)], axis=1)
    anchors = (shifts[:, None, :] + base[None, :, :]).reshape(-1, 4)
    wa = anchors[:, 2] - anchors[:, 0]
    ha = anchors[:, 3] - anchors[:, 1]
    cxa = anchors[:, 0] + 0.5 * wa
    cya = anchors[:, 1] + 0.5 * ha
    out = []
    for v in (wa, ha, cxa, cya):
        out.append(jnp.pad(v, (0, PADN - NUM)).reshape(ROWS, LANES))
    return out


def _nms_kernel(keys_ref, dx_ref, dy_ref, dw_ref, dh_ref,
                wa_ref, ha_ref, cxa_ref, cya_ref,
                out_ref,
                sw_ref, x1_ref, y1_ref, x2_ref, y2_ref, a2_ref):
    idx = jax.lax.broadcasted_iota(jnp.int32, (ROWS, LANES), 0) * LANES + \
        jax.lax.broadcasted_iota(jnp.int32, (ROWS, LANES), 1)

    for a in range(N):
        lg = keys_ref[a, :, :]

        u = jax.lax.bitcast_convert_type(lg, jnp.int32)
        key = jnp.where(u >= 0, u, jnp.int32(-2147483648) - u)

        def tbody(_, lohi, key=key):
            lo, hi = lohi
            mid = (lo >> 1) + (hi >> 1) + (lo & hi & 1)
            cnt = jnp.sum((key >= mid).astype(jnp.int32))
            big = cnt >= PRE_NMS
            return (jnp.where(big, mid, lo), jnp.where(big, hi, mid))

        lo, _ = jax.lax.fori_loop(
            0, 32, tbody, (jnp.int32(-2147483648), jnp.int32(2147483647)))
        tie = key == lo
        need = PRE_NMS - jnp.sum((key > lo).astype(jnp.int32))

        def ibody(_, lohi, tie=tie, need=need):
            ilo, ihi = lohi
            mid = (ilo + ihi) // 2
            cnt = jnp.sum((tie & (idx < mid)).astype(jnp.int32))
            small = cnt < need
            return (jnp.where(small, mid, ilo), jnp.where(small, ihi, mid))

        _, istar = jax.lax.fori_loop(0, 17, ibody, (jnp.int32(0), jnp.int32(PADN)))
        sel = (key > lo) | (tie & (idx < istar))

        wa = wa_ref[...]
        ha = ha_ref[...]
        dw = jnp.minimum(dw_ref[a, :, :], BBOX_XFORM_CLIP)
        dh = jnp.minimum(dh_ref[a, :, :], BBOX_XFORM_CLIP)
        pcx = dx_ref[a, :, :] * wa + cxa_ref[...]
        pcy = dy_ref[a, :, :] * ha + cya_ref[...]
        pw = jnp.exp(dw) * wa
        ph = jnp.exp(dh) * ha
        x1 = jnp.clip(pcx - 0.5 * pw, 0.0, float(IMG_W))
        y1 = jnp.clip(pcy - 0.5 * ph, 0.0, float(IMG_H))
        x2 = jnp.clip(pcx + 0.5 * pw, 0.0, float(IMG_W))
        y2 = jnp.clip(pcy + 0.5 * ph, 0.0, float(IMG_H))
        keep = ((x2 - x1) >= 0.0) & ((y2 - y1) >= 0.0)

        sw_ref[a, :, :] = jnp.where(sel & keep, lg, _NEG)
        x1_ref[a, :, :] = x1
        y1_ref[a, :, :] = y1
        x2_ref[a, :, :] = x2
        y2_ref[a, :, :] = y2
        a2_ref[a, :, :] = (x2 - x1) * (y2 - y1)

    lane = jax.lax.broadcasted_iota(jnp.int32, (1, LANES), 1)

    def body(i, carry):
        for a in range(N):
            sw = sw_ref[a, :, :]
            m = jnp.max(sw)
            valid = m > -1e9
            istar = jnp.min(jnp.where(sw == m, idx, jnp.int32(PADN)))
            oh = idx == istar
            ohf = oh.astype(jnp.float32)
            x1 = x1_ref[a, :, :]
            y1 = y1_ref[a, :, :]
            x2 = x2_ref[a, :, :]
            y2 = y2_ref[a, :, :]
            bx1 = jnp.sum(x1 * ohf)
            by1 = jnp.sum(y1 * ohf)
            bx2 = jnp.sum(x2 * ohf)
            by2 = jnp.sum(y2 * ohf)
            ba = (bx2 - bx1) * (by2 - by1)
            iw = jnp.maximum(jnp.minimum(bx2, x2) - jnp.maximum(bx1, x1), 0.0)
            ih = jnp.maximum(jnp.minimum(by2, y2) - jnp.maximum(by1, y1), 0.0)
            inter = iw * ih
            iou = inter / (ba + a2_ref[a, :, :] - inter + 1e-9)
            sup = (iou > NMS_THRESH) | oh
            sw_ref[a, :, :] = jnp.where(sup & valid, _NEG, sw)
            vf = jnp.where(valid, 1.0, 0.0)
            row = jnp.where(lane == 0, jnp.where(valid, bx1, 0.0),
                  jnp.where(lane == 1, jnp.where(valid, by1, 0.0),
                  jnp.where(lane == 2, jnp.where(valid, bx2, 0.0),
                  jnp.where(lane == 3, jnp.where(valid, by2, 0.0),
                  jnp.where(lane == 4, vf, 0.0)))))
            out_ref[a, pl.ds(i, 1), :] = row
        return carry

    jax.lax.fori_loop(0, POST_NMS, body, 0)


def _run_nms(logits_flat, breg_flat):
    padk = jnp.pad(logits_flat, ((0, 0), (0, PADN - NUM)),
                   constant_values=_NEG).reshape(N, ROWS, LANES)
    regs = []
    for j in range(4):
        regs.append(jnp.pad(breg_flat[:, :, j], ((0, 0), (0, PADN - NUM))
                            ).reshape(N, ROWS, LANES))
    wa, ha, cxa, cya = _anchor_planes()

    img_spec = pl.BlockSpec((N, ROWS, LANES), lambda: (0, 0, 0))
    cst_spec = pl.BlockSpec((ROWS, LANES), lambda: (0, 0))
    out = pl.pallas_call(
        _nms_kernel,
        grid=(),
        in_specs=[img_spec] * 5 + [cst_spec] * 4,
        out_specs=pl.BlockSpec((N, OUT_ROWS, LANES), lambda: (0, 0, 0)),
        out_shape=jax.ShapeDtypeStruct((N, OUT_ROWS, LANES), jnp.float32),
        scratch_shapes=[pltpu.VMEM((N, ROWS, LANES), jnp.float32)] * 6,
    )(padk, *regs, wa, ha, cxa, cya)

    boxes = out[:, :POST_NMS, :4]
    valid = out[:, :POST_NMS, 4] > 0.5
    return boxes, valid


WP = W + 2
PGRID = (H + 2) * WP
MROWS = 4480
XROWS = 4680
K_IM2COL = 9 * C_IN


def _trunk_kernel(x_ref, w_ref, b_ref, wh_ref, bh_ref, out_ref):
    t = None
    for k in range(9):
        kh, kw = k // 3, k % 3
        start = 88 + (kh - 1) * WP + (kw - 1)
        sl = x_ref[0, pl.ds(start, MROWS), :]
        d = jnp.dot(sl, w_ref[k * C_IN:(k + 1) * C_IN, :],
                    preferred_element_type=jnp.float32)
        t = d if t is None else t + d
    t = jax.nn.relu(t + b_ref[...])
    o = jnp.dot(t, wh_ref[...], preferred_element_type=jnp.float32)
    out_ref[0, :, :] = o + bh_ref[...]


def _trunk(features, conv_w, conv_b, cls_w, cls_b, bbox_w, bbox_b):
    ft = features.transpose(0, 2, 3, 1)
    ft = jnp.pad(ft, ((0, 0), (1, 1), (1, 1), (0, 0))).reshape(N, PGRID, C_IN)
    x = jnp.pad(ft, ((0, 0), (88, XROWS - 88 - PGRID), (0, 0)))
    w = conv_w.transpose(2, 3, 1, 0).reshape(K_IM2COL, C_MID)
    wh = jnp.zeros((C_MID, 256), jnp.float32)
    wh = wh.at[:, :A].set(cls_w[:, :, 0, 0].T)
    wh = wh.at[:, 128:128 + 4 * A].set(bbox_w[:, :, 0, 0].T)
    bh = jnp.zeros((256,), jnp.float32)
    bh = bh.at[:A].set(cls_b)
    bh = bh.at[128:128 + 4 * A].set(bbox_b)
    b2 = conv_b.reshape(1, C_MID)
    bh2 = bh.reshape(1, 256)

    out = pl.pallas_call(
        _trunk_kernel,
        grid=(N,),
        in_specs=[
            pl.BlockSpec((1, XROWS, C_IN), lambda i: (i, 0, 0)),
            pl.BlockSpec((K_IM2COL, C_MID), lambda i: (0, 0)),
            pl.BlockSpec((1, C_MID), lambda i: (0, 0)),
            pl.BlockSpec((C_MID, 256), lambda i: (0, 0)),
            pl.BlockSpec((1, 256), lambda i: (0, 0)),
        ],
        out_specs=pl.BlockSpec((1, MROWS, 256), lambda i: (i, 0, 0)),
        out_shape=jax.ShapeDtypeStruct((N, MROWS, 256), jnp.float32),
        compiler_params=pltpu.CompilerParams(
            dimension_semantics=("arbitrary",)),
    )(x, w, b2, wh, bh2)
    return out


def _conv2d(x, w, b):
    y = jax.lax.conv_general_dilated(
        x, w, (1, 1), "SAME", dimension_numbers=("NCHW", "OIHW", "NCHW"))
    return y + b[None, :, None, None]


def kernel(images, features, img_metas, conv_w, conv_b, cls_w, cls_b, bbox_w, bbox_b):
    trunk = _trunk(features, conv_w, conv_b, cls_w, cls_b, bbox_w, bbox_b)
    yy, xx = jnp.meshgrid(jnp.arange(H), jnp.arange(W), indexing="ij")
    rows = ((yy + 1) * WP + (xx + 1)).reshape(-1)
    tv = trunk[:, rows, :]
    logits = tv[:, :, :A].transpose(0, 2, 1).reshape(N, A, H, W)

    logits_flat = tv[:, :, :A].reshape(N, NUM)
    breg_flat = tv[:, :, 128:128 + 4 * A].reshape(N, NUM, 4)

    scores_flat = jax.nn.sigmoid(logits_flat)
    boxes, valid = _run_nms(scores_flat, breg_flat)
    return boxes, valid, logits


# speedup vs baseline: 1.3374x; 1.0540x over previous
import jax
import jax.numpy as jnp
import numpy as np
from jax.experimental import pallas as pl
from jax.experimental.pallas import tpu as pltpu

N = 2
C_IN = 256
C_MID = 256
H = 50
W = 84
STRIDE = 16
SCALES = (32.0, 64.0, 128.0, 256.0, 512.0)
RATIOS = (0.5, 1.0, 2.0)
A = len(SCALES) * len(RATIOS)
IMG_W = 1344
IMG_H = 800
PRE_NMS = 6000
POST_NMS = 1000
NMS_THRESH = 0.7
BBOX_XFORM_CLIP = float(np.log(1000.0 / 16.0))

NUM = A * H * W
LANES = 128
ROWS = 496
PADN = ROWS * LANES
OUT_ROWS = 1024

_NEG = -1e10


def _anchor_planes():
    scales = jnp.asarray(SCALES, jnp.float32)
    ratios = jnp.asarray(RATIOS, jnp.float32)
    h_ratios = jnp.sqrt(ratios)
    w_ratios = 1.0 / h_ratios
    ws = (w_ratios[:, None] * scales[None, :]).reshape(-1)
    hs = (h_ratios[:, None] * scales[None, :]).reshape(-1)
    base = jnp.stack([-ws, -hs, ws, hs], axis=1) / 2.0
    sx = jnp.arange(W, dtype=jnp.float32) * STRIDE
    sy = jnp.arange(H, dtype=jnp.float32) * STRIDE
    yy, xx = jnp.meshgrid(sy, sx, indexing="ij")
    shifts = jnp.stack([xx.reshape(-1), yy.reshape(-1), xx.reshape(-1), yy.reshape(-1)], axis=1)
    anchors = (shifts[:, None, :] + base[None, :, :]).reshape(-1, 4)
    wa = anchors[:, 2] - anchors[:, 0]
    ha = anchors[:, 3] - anchors[:, 1]
    cxa = anchors[:, 0] + 0.5 * wa
    cya = anchors[:, 1] + 0.5 * ha
    out = []
    for v in (wa, ha, cxa, cya):
        out.append(jnp.pad(v, (0, PADN - NUM)).reshape(ROWS, LANES))
    return out


def _nms_kernel(keys_ref, dx_ref, dy_ref, dw_ref, dh_ref,
                wa_ref, ha_ref, cxa_ref, cya_ref,
                out_ref,
                sw_ref, x1_ref, y1_ref, x2_ref, y2_ref, a2_ref):
    idx = jax.lax.broadcasted_iota(jnp.int32, (ROWS, LANES), 0) * LANES + \
        jax.lax.broadcasted_iota(jnp.int32, (ROWS, LANES), 1)

    for a in range(N):
        lg = keys_ref[a, :, :]

        u = jax.lax.bitcast_convert_type(lg, jnp.int32)
        key = jnp.where(u >= 0, u, jnp.int32(-2147483648) - u)

        def tbody(_, lohi, key=key):
            lo, hi = lohi
            mid = (lo >> 1) + (hi >> 1) + (lo & hi & 1)
            cnt = jnp.sum((key >= mid).astype(jnp.int32))
            big = cnt >= PRE_NMS
            return (jnp.where(big, mid, lo), jnp.where(big, hi, mid))

        lo, _ = jax.lax.fori_loop(
            0, 32, tbody, (jnp.int32(-2147483648), jnp.int32(2147483647)))
        tie = key == lo
        need = PRE_NMS - jnp.sum((key > lo).astype(jnp.int32))

        def ibody(_, lohi, tie=tie, need=need):
            ilo, ihi = lohi
            mid = (ilo + ihi) // 2
            cnt = jnp.sum((tie & (idx < mid)).astype(jnp.int32))
            small = cnt < need
            return (jnp.where(small, mid, ilo), jnp.where(small, ihi, mid))

        _, istar = jax.lax.fori_loop(0, 17, ibody, (jnp.int32(0), jnp.int32(PADN)))
        sel = (key > lo) | (tie & (idx < istar))

        wa = wa_ref[...]
        ha = ha_ref[...]
        dw = jnp.minimum(dw_ref[a, :, :], BBOX_XFORM_CLIP)
        dh = jnp.minimum(dh_ref[a, :, :], BBOX_XFORM_CLIP)
        pcx = dx_ref[a, :, :] * wa + cxa_ref[...]
        pcy = dy_ref[a, :, :] * ha + cya_ref[...]
        pw = jnp.exp(dw) * wa
        ph = jnp.exp(dh) * ha
        x1 = jnp.clip(pcx - 0.5 * pw, 0.0, float(IMG_W))
        y1 = jnp.clip(pcy - 0.5 * ph, 0.0, float(IMG_H))
        x2 = jnp.clip(pcx + 0.5 * pw, 0.0, float(IMG_W))
        y2 = jnp.clip(pcy + 0.5 * ph, 0.0, float(IMG_H))
        keep = ((x2 - x1) >= 0.0) & ((y2 - y1) >= 0.0)

        sw_ref[a, :, :] = jnp.where(sel & keep, lg, _NEG)
        x1_ref[a, :, :] = x1
        y1_ref[a, :, :] = y1
        x2_ref[a, :, :] = x2
        y2_ref[a, :, :] = y2
        a2_ref[a, :, :] = (x2 - x1) * (y2 - y1)

    lane = jax.lax.broadcasted_iota(jnp.int32, (1, LANES), 1)

    zrow = jnp.zeros((1, LANES), jnp.float32)

    def body(i, carry):
        for a in range(N):
            sw = sw_ref[a, :, :]
            m = jnp.max(sw)
            valid = m > -1e9
            istar = jnp.min(jnp.where(sw == m, idx, jnp.int32(PADN)))
            r = istar >> 7
            c = istar & 127
            out_ref[a, pl.ds(i, 1), :] = zrow

            @pl.when(valid)
            def _(a=a, r=r, c=c, i=i):
                lc = lane == c
                bx1 = jnp.max(jnp.where(lc, x1_ref[a, pl.ds(r, 1), :], 0.0))
                by1 = jnp.max(jnp.where(lc, y1_ref[a, pl.ds(r, 1), :], 0.0))
                bx2 = jnp.max(jnp.where(lc, x2_ref[a, pl.ds(r, 1), :], 0.0))
                by2 = jnp.max(jnp.where(lc, y2_ref[a, pl.ds(r, 1), :], 0.0))
                ba = (bx2 - bx1) * (by2 - by1)
                x1 = x1_ref[a, :, :]
                y1 = y1_ref[a, :, :]
                x2 = x2_ref[a, :, :]
                y2 = y2_ref[a, :, :]
                iw = jnp.maximum(jnp.minimum(bx2, x2) - jnp.maximum(bx1, x1), 0.0)
                ih = jnp.maximum(jnp.minimum(by2, y2) - jnp.maximum(by1, y1), 0.0)
                inter = iw * ih
                iou = inter / (ba + a2_ref[a, :, :] - inter + 1e-9)
                sw = sw_ref[a, :, :]
                sw_ref[a, :, :] = jnp.where(iou > NMS_THRESH, _NEG, sw)
                swr = sw_ref[a, pl.ds(r, 1), :]
                sw_ref[a, pl.ds(r, 1), :] = jnp.where(lc, _NEG, swr)
                row = jnp.where(lane == 0, bx1,
                      jnp.where(lane == 1, by1,
                      jnp.where(lane == 2, bx2,
                      jnp.where(lane == 3, by2,
                      jnp.where(lane == 4, 1.0, 0.0)))))
                out_ref[a, pl.ds(i, 1), :] = row
        return carry

    jax.lax.fori_loop(0, POST_NMS, body, 0)


def _run_nms(logits_flat, breg_flat):
    padk = jnp.pad(logits_flat, ((0, 0), (0, PADN - NUM)),
                   constant_values=_NEG).reshape(N, ROWS, LANES)
    regs = []
    for j in range(4):
        regs.append(jnp.pad(breg_flat[:, :, j], ((0, 0), (0, PADN - NUM))
                            ).reshape(N, ROWS, LANES))
    wa, ha, cxa, cya = _anchor_planes()

    img_spec = pl.BlockSpec((N, ROWS, LANES), lambda: (0, 0, 0))
    cst_spec = pl.BlockSpec((ROWS, LANES), lambda: (0, 0))
    out = pl.pallas_call(
        _nms_kernel,
        grid=(),
        in_specs=[img_spec] * 5 + [cst_spec] * 4,
        out_specs=pl.BlockSpec((N, OUT_ROWS, LANES), lambda: (0, 0, 0)),
        out_shape=jax.ShapeDtypeStruct((N, OUT_ROWS, LANES), jnp.float32),
        scratch_shapes=[pltpu.VMEM((N, ROWS, LANES), jnp.float32)] * 6,
    )(padk, *regs, wa, ha, cxa, cya)

    boxes = out[:, :POST_NMS, :4]
    valid = out[:, :POST_NMS, 4] > 0.5
    return boxes, valid


WP = W + 2
PGRID = (H + 2) * WP
MROWS = 4480
XROWS = 4680
K_IM2COL = 9 * C_IN


def _trunk_kernel(x_ref, w_ref, b_ref, wh_ref, bh_ref, out_ref):
    t = None
    for k in range(9):
        kh, kw = k // 3, k % 3
        start = 88 + (kh - 1) * WP + (kw - 1)
        sl = x_ref[0, pl.ds(start, MROWS), :]
        d = jnp.dot(sl, w_ref[k * C_IN:(k + 1) * C_IN, :],
                    preferred_element_type=jnp.float32)
        t = d if t is None else t + d
    t = jax.nn.relu(t + b_ref[...])
    o = jnp.dot(t, wh_ref[...], preferred_element_type=jnp.float32)
    out_ref[0, :, :] = o + bh_ref[...]


def _trunk(features, conv_w, conv_b, cls_w, cls_b, bbox_w, bbox_b):
    ft = features.transpose(0, 2, 3, 1)
    ft = jnp.pad(ft, ((0, 0), (1, 1), (1, 1), (0, 0))).reshape(N, PGRID, C_IN)
    x = jnp.pad(ft, ((0, 0), (88, XROWS - 88 - PGRID), (0, 0)))
    w = conv_w.transpose(2, 3, 1, 0).reshape(K_IM2COL, C_MID)
    wh = jnp.zeros((C_MID, 256), jnp.float32)
    wh = wh.at[:, :A].set(cls_w[:, :, 0, 0].T)
    wh = wh.at[:, 128:128 + 4 * A].set(bbox_w[:, :, 0, 0].T)
    bh = jnp.zeros((256,), jnp.float32)
    bh = bh.at[:A].set(cls_b)
    bh = bh.at[128:128 + 4 * A].set(bbox_b)
    b2 = conv_b.reshape(1, C_MID)
    bh2 = bh.reshape(1, 256)

    out = pl.pallas_call(
        _trunk_kernel,
        grid=(N,),
        in_specs=[
            pl.BlockSpec((1, XROWS, C_IN), lambda i: (i, 0, 0)),
            pl.BlockSpec((K_IM2COL, C_MID), lambda i: (0, 0)),
            pl.BlockSpec((1, C_MID), lambda i: (0, 0)),
            pl.BlockSpec((C_MID, 256), lambda i: (0, 0)),
            pl.BlockSpec((1, 256), lambda i: (0, 0)),
        ],
        out_specs=pl.BlockSpec((1, MROWS, 256), lambda i: (i, 0, 0)),
        out_shape=jax.ShapeDtypeStruct((N, MROWS, 256), jnp.float32),
        compiler_params=pltpu.CompilerParams(
            dimension_semantics=("arbitrary",)),
    )(x, w, b2, wh, bh2)
    return out


def _conv2d(x, w, b):
    y = jax.lax.conv_general_dilated(
        x, w, (1, 1), "SAME", dimension_numbers=("NCHW", "OIHW", "NCHW"))
    return y + b[None, :, None, None]


def kernel(images, features, img_metas, conv_w, conv_b, cls_w, cls_b, bbox_w, bbox_b):
    trunk = _trunk(features, conv_w, conv_b, cls_w, cls_b, bbox_w, bbox_b)
    yy, xx = jnp.meshgrid(jnp.arange(H), jnp.arange(W), indexing="ij")
    rows = ((yy + 1) * WP + (xx + 1)).reshape(-1)
    tv = trunk[:, rows, :]
    logits = tv[:, :, :A].transpose(0, 2, 1).reshape(N, A, H, W)

    logits_flat = tv[:, :, :A].reshape(N, NUM)
    breg_flat = tv[:, :, 128:128 + 4 * A].reshape(N, NUM, 4)

    scores_flat = jax.nn.sigmoid(logits_flat)
    boxes, valid = _run_nms(scores_flat, breg_flat)
    return boxes, valid, logits
